# Initial kernel scaffold; baseline (speedup 1.0000x reference)
import jax
import jax.numpy as jnp
from jax.experimental import pallas as pl


def kernel(x, edge_index, edge_feature, K_v2v, K_e2v, V_v2v, V_e2v, lu_W, lu_b, ln1_g, ln1_b, msg_W, msg_b, ln2_g, ln2_b, cat_W, cat_b, bn_g, bn_b):
    raise NotImplementedError("write your pallas kernel here")


# speedup vs baseline: 3.3305x; 3.3305x over previous
import functools

import jax
import jax.numpy as jnp
import numpy as np
from jax import lax
from jax.experimental import pallas as pl
from jax.experimental.pallas import tpu as pltpu
from jax.experimental.pallas import tpu_sc as plsc

N = 10000
E = 160000
D = 128
HEAD = 2

CH = 128
E_PAD = 163840
GT = 32
GC = E_PAD // (GT * CH)
ST = 16
SC_CHUNKS = E_PAD // (ST * CH)
NPAD = 10240
NT = NPAD // ST

@functools.lru_cache(maxsize=None)
def _sc_kernels():
    mesh = plsc.VectorSubcoreMesh(
        core_axis_name="c", subcore_axis_name="s",
        num_cores=2, num_subcores=16)

    @functools.partial(
        pl.kernel,
        out_type=(
            jax.ShapeDtypeStruct((E_PAD, D), jnp.float32),
            jax.ShapeDtypeStruct((E_PAD, D), jnp.float32),
        ),
        mesh=mesh,
        scratch_types=[
            pltpu.VMEM((GC, CH), jnp.int32),
            pltpu.VMEM((GC, CH), jnp.int32),
            pltpu.VMEM((CH, D), jnp.float32),
            pltpu.VMEM((CH, D), jnp.float32),
            pltpu.SemaphoreType.DMA,
            pltpu.SemaphoreType.DMA,
        ],
    )
    def gather_xe(x_hbm, src_hbm, dst_hbm, xs_out, xd_out,
                  src_v, dst_v, buf_s, buf_d, sem_s, sem_d):
        wid = lax.axis_index("s") * 2 + lax.axis_index("c")
        base = wid * (GC * CH)
        pltpu.sync_copy(src_hbm.at[wid], src_v)
        pltpu.sync_copy(dst_hbm.at[wid], dst_v)

        def body(j, carry):
            cs = pltpu.async_copy(x_hbm.at[src_v.at[j]], buf_s, sem_s)
            cd = pltpu.async_copy(x_hbm.at[dst_v.at[j]], buf_d, sem_d)
            cs.wait()
            pltpu.sync_copy(buf_s, xs_out.at[pl.ds(base + j * CH, CH)])
            cd.wait()
            pltpu.sync_copy(buf_d, xd_out.at[pl.ds(base + j * CH, CH)])
            return carry

        lax.fori_loop(0, GC, body, 0)

    @functools.partial(
        pl.kernel,
        out_type=jax.ShapeDtypeStruct((HEAD, NPAD, D), jnp.float32),
        mesh=mesh,
        scratch_types=[
            pltpu.VMEM((SC_CHUNKS, CH), jnp.int32),
            pltpu.VMEM((CH, D), jnp.float32),
            pltpu.VMEM_SHARED((NPAD, D), jnp.float32),
            pltpu.SemaphoreType.DMA,
        ],
    )
    def scatter_agg(msg_hbm, dst_hbm, zeros_hbm, out_hbm, idx_v, buf, acc, sem):
        c = lax.axis_index("c")
        s = lax.axis_index("s")
        base = s * (SC_CHUNKS * CH)
        pltpu.sync_copy(zeros_hbm, buf)

        def zbody(i, carry):
            pltpu.sync_copy(buf, acc.at[pl.ds(s * NT + i * CH, CH)])
            return carry

        lax.fori_loop(0, NT // CH, zbody, 0)
        pltpu.sync_copy(dst_hbm.at[s], idx_v)
        plsc.subcore_barrier()

        def body(j, carry):
            pltpu.sync_copy(msg_hbm.at[c, pl.ds(base + j * CH, CH)], buf)
            pltpu.sync_copy(buf, acc.at[idx_v.at[j]], add=True)
            return carry

        lax.fori_loop(0, SC_CHUNKS, body, 0)
        plsc.subcore_barrier()

        def obody(i, carry):
            pltpu.sync_copy(acc.at[pl.ds(s * NT + i * CH, CH)], buf)
            pltpu.sync_copy(buf, out_hbm.at[c, pl.ds(s * NT + i * CH, CH)])
            return carry

        lax.fori_loop(0, NT // CH, obody, 0)

    return gather_xe, scatter_agg


BE = 1024

_SCALE = float(1.0 / np.sqrt(np.float32(2 * D)))


def _ln(v, g, b):
    m = jnp.mean(v, axis=-1, keepdims=True)
    m2 = jnp.mean(v * v, axis=-1, keepdims=True)
    var = m2 - m * m
    return (v - m) * lax.rsqrt(var + 1e-5) * g + b


def _leaky(v):
    return jnp.where(v >= 0, v, 0.01 * v)


def _prep_body(Kv_ref, Ke_ref, Vv_ref, Ve_ref, luW_ref, msgW_ref,
               wxs_ref, wef_ref, wmsg_ref):
    bf = jnp.bfloat16
    f32 = jnp.float32
    for h in range(HEAD):
        wxs_ref[:, h * D:(h + 1) * D] = Kv_ref[h].astype(bf)
        wef_ref[:, h * D:(h + 1) * D] = Ke_ref[h].astype(bf)
        a = jnp.dot(Vv_ref[h], luW_ref[h, :D, :], preferred_element_type=f32)
        b = jnp.dot(Ve_ref[h], luW_ref[h, D:, :], preferred_element_type=f32)
        lo = 2 * D + 2 * h * D
        wxs_ref[:, lo:lo + 2 * D] = a.astype(bf)
        wef_ref[:, lo:lo + 2 * D] = b.astype(bf)
        wmsg_ref[h] = msgW_ref[h].astype(bf)


def _prep_weights(K_v2v, K_e2v, V_v2v, V_e2v, lu_W, msg_W):
    full = lambda shape: pl.BlockSpec(shape, lambda *_: (0,) * len(shape))
    return pl.pallas_call(
        _prep_body,
        in_specs=[
            full((HEAD, D, D)),
            full((HEAD, D, D)),
            full((HEAD, D, D)),
            full((HEAD, D, D)),
            full((HEAD, 2 * D, 2 * D)),
            full((HEAD, 2 * D, D)),
        ],
        out_specs=(full((D, 6 * D)), full((D, 6 * D)), full((HEAD, 2 * D, D))),
        out_shape=(
            jax.ShapeDtypeStruct((D, 6 * D), jnp.bfloat16),
            jax.ShapeDtypeStruct((D, 6 * D), jnp.bfloat16),
            jax.ShapeDtypeStruct((HEAD, 2 * D, D), jnp.bfloat16),
        ),
    )(K_v2v, K_e2v, V_v2v, V_e2v, lu_W, msg_W)


def _edge_body(xs_ref, xd_ref, ef_ref, wxs_ref, wef_ref, wmsg_ref,
               lub_ref, ln1g_ref, ln1b_ref, msgb_ref, ln2g_ref, ln2b_ref,
               out_ref):
    bf = jnp.bfloat16
    f32 = jnp.float32
    xsb = xs_ref[...].astype(bf)
    xdb = xd_ref[...].astype(bf)
    efb = ef_ref[...].astype(bf)
    s = jnp.dot(xsb, wxs_ref[...], preferred_element_type=f32)
    qq = jnp.dot(xdb, wxs_ref[:, :2 * D], preferred_element_type=f32)
    eo = jnp.dot(efb, wef_ref[...], preferred_element_type=f32)
    for h in range(HEAD):
        q = qq[:, h * D:(h + 1) * D]
        k = s[:, h * D:(h + 1) * D]
        ke = eo[:, h * D:(h + 1) * D]
        alpha = jnp.concatenate([q * k, q * ke], axis=1) * _SCALE
        gate = _ln(alpha, ln1g_ref[h:h + 1, :], ln1b_ref[h:h + 1, :])
        gate = 1.0 / (1.0 + jnp.exp(-gate))
        lo = 2 * D + 2 * h * D
        m = (s[:, lo:lo + 2 * D] + eo[:, lo:lo + 2 * D]
             + lub_ref[h:h + 1, :]) * gate
        m = (jnp.dot(m.astype(bf), wmsg_ref[h], preferred_element_type=f32)
             + msgb_ref[h:h + 1, :])
        out_ref[h] = _leaky(_ln(m, ln2g_ref[h:h + 1, :], ln2b_ref[h:h + 1, :]))


def _edge_msgs(xs, xd, ef, wxs, wef, wmsg, lu_b,
               ln1_g, ln1_b, msg_b, ln2_g, ln2_b):
    grid = E_PAD // BE
    full = lambda shape: pl.BlockSpec(shape, lambda i: (0,) * len(shape))
    return pl.pallas_call(
        _edge_body,
        grid=(grid,),
        in_specs=[
            pl.BlockSpec((BE, D), lambda i: (i, 0)),
            pl.BlockSpec((BE, D), lambda i: (i, 0)),
            pl.BlockSpec((BE, D), lambda i: (i, 0)),
            full((D, 6 * D)),
            full((D, 6 * D)),
            full((HEAD, 2 * D, D)),
            full((HEAD, 2 * D)),
            full((HEAD, 2 * D)),
            full((HEAD, 2 * D)),
            full((HEAD, D)),
            full((HEAD, D)),
            full((HEAD, D)),
        ],
        out_specs=pl.BlockSpec((HEAD, BE, D), lambda i: (0, i, 0)),
        out_shape=jax.ShapeDtypeStruct((HEAD, E_PAD, D), jnp.float32),
    )(xs, xd, ef, wxs, wef, wmsg, lu_b, ln1_g, ln1_b, msg_b, ln2_g, ln2_b)


def _final_body(agg_ref, x_ref, catW_ref, catb_ref, bng_ref, bnb_ref, out_ref):
    h0 = agg_ref[0, :N, :]
    h1 = agg_ref[1, :N, :]
    out = (jnp.dot(h0, catW_ref[0], preferred_element_type=jnp.float32)
           + jnp.dot(h1, catW_ref[1], preferred_element_type=jnp.float32)
           + catb_ref[0:1, :])
    mean = jnp.mean(out, axis=0, keepdims=True)
    var = jnp.mean((out - mean) ** 2, axis=0, keepdims=True)
    out = (out - mean) * lax.rsqrt(var + 1e-5) * bng_ref[0:1, :] + bnb_ref[0:1, :]
    out_ref[...] = _leaky(out) + x_ref[...]


def _finalize(agg, x, cat_W, cat_b, bn_g, bn_b):
    full = lambda shape: pl.BlockSpec(shape, lambda *_: (0,) * len(shape))
    return pl.pallas_call(
        _final_body,
        in_specs=[
            full((HEAD, NPAD, D)),
            full((N, D)),
            full((HEAD, D, D)),
            full((1, D)),
            full((1, D)),
            full((1, D)),
        ],
        out_specs=full((N, D)),
        out_shape=jax.ShapeDtypeStruct((N, D), jnp.float32),
    )(agg, x, cat_W.reshape(HEAD, D, D), cat_b.reshape(1, D),
      bn_g.reshape(1, D), bn_b.reshape(1, D))


def kernel(x, edge_index, edge_feature, K_v2v, K_e2v, V_v2v, V_e2v,
           lu_W, lu_b, ln1_g, ln1_b, msg_W, msg_b, ln2_g, ln2_b,
           cat_W, cat_b, bn_g, bn_b):
    pad = E_PAD - E
    src = jnp.concatenate([edge_index[0], jnp.zeros((pad,), jnp.int32)])
    dst = edge_index[1]
    dst_g = jnp.concatenate([dst, jnp.zeros((pad,), jnp.int32)])
    dst_s = jnp.concatenate([dst, jnp.full((pad,), N, jnp.int32)])
    ef = jnp.concatenate(
        [edge_feature, jnp.zeros((pad, D), jnp.float32)], axis=0)

    gather_xe, scatter_agg = _sc_kernels()
    xs, xd = gather_xe(x,
                       src.reshape(GT, GC, CH),
                       dst_g.reshape(GT, GC, CH))
    wxs, wef, wmsg = _prep_weights(K_v2v, K_e2v, V_v2v, V_e2v, lu_W, msg_W)
    msg = _edge_msgs(xs, xd, ef, wxs, wef, wmsg, lu_b,
                     ln1_g, ln1_b, msg_b, ln2_g, ln2_b)
    agg = scatter_agg(msg,
                      dst_s.reshape(ST, SC_CHUNKS, CH),
                      jnp.zeros((CH, D), jnp.float32))
    return _finalize(agg, x, cat_W, cat_b, bn_g, bn_b)
